# Initial kernel scaffold; baseline (speedup 1.0000x reference)
import jax
import jax.numpy as jnp
from jax.experimental import pallas as pl


def kernel(x, edge_index, params):
    raise NotImplementedError("write your pallas kernel here")


# speedup vs baseline: 17.0987x; 17.0987x over previous
import functools

import jax
import jax.numpy as jnp
from jax import lax
from jax.experimental import pallas as pl
from jax.experimental.pallas import tpu as pltpu
from jax.experimental.pallas import tpu_sc as plsc

_NC = 2
_NS = 16
_NW = _NC * _NS
_D = 8
_C = 128


@functools.cache
def _make_sc_agg(n_nodes: int, n_edges: int):
    epw = n_edges // _NW
    nfull = epw // _C
    tail = epw - nfull * _C
    rps = n_nodes // _NS

    mesh = plsc.VectorSubcoreMesh(core_axis_name="c", subcore_axis_name="s")

    @functools.partial(
        pl.kernel,
        mesh=mesh,
        out_type=jax.ShapeDtypeStruct((_NC, n_nodes, _D), jnp.float32),
        compiler_params=pltpu.CompilerParams(use_tc_tiling_on_sc=False),
        scratch_types=[
            pltpu.VMEM_SHARED((n_nodes, _D), jnp.float32),
            pltpu.VMEM_SHARED((n_nodes, _D), jnp.float32),
            pltpu.VMEM((_C,), jnp.int32),
            pltpu.VMEM((_C,), jnp.int32),
            pltpu.VMEM((_C, _D), jnp.float32),
            pltpu.VMEM((tail or 8,), jnp.int32),
            pltpu.VMEM((tail or 8,), jnp.int32),
            pltpu.VMEM((tail or 8, _D), jnp.float32),
        ],
    )
    def sc_agg(y_hbm, src_hbm, dst_hbm, zeros_hbm, out_hbm,
               ytab, acc, srcv, dstv, rows, srcv_t, dstv_t, rows_t):
        c = lax.axis_index("c")
        s = lax.axis_index("s")
        wid = c * _NS + s

        r0 = s * rps
        pltpu.sync_copy(y_hbm.at[pl.ds(r0, rps), :], ytab.at[pl.ds(r0, rps), :])
        pltpu.sync_copy(zeros_hbm.at[pl.ds(r0, rps), :], acc.at[pl.ds(r0, rps), :])
        plsc.subcore_barrier()

        ebase = wid * epw

        def chunk(off, cw, sv, dv, rv):
            pltpu.sync_copy(src_hbm.at[pl.ds(off, cw)], sv)
            pltpu.sync_copy(dst_hbm.at[pl.ds(off, cw)], dv)
            pltpu.sync_copy(ytab.at[sv], rv)
            pltpu.sync_copy(rv, acc.at[dv], add=True)

        def body(j, carry):
            chunk(ebase + j * _C, _C, srcv, dstv, rows)
            return carry

        lax.fori_loop(0, nfull, body, 0)
        if tail:
            chunk(ebase + nfull * _C, tail, srcv_t, dstv_t, rows_t)

        plsc.subcore_barrier()
        pltpu.sync_copy(acc.at[pl.ds(r0, rps), :],
                        out_hbm.at[c, pl.ds(r0, rps), :])

    return sc_agg


def kernel(x, edge_index, params):
    n = x.shape[0]
    e = edge_index.shape[1]
    npad = -(-n // 128) * 128
    src = edge_index[0]
    dst = edge_index[1]
    sc_agg = _make_sc_agg(npad, e)
    zeros = jnp.zeros((npad, _D), jnp.float32)

    def pad8(m):
        return jnp.pad(m, ((0, npad - n), (0, _D - m.shape[1])))

    def mean_agg(y):
        parts = sc_agg(y, src, dst, zeros)
        return parts[0, :n] + parts[1, :n]

    sa = mean_agg(pad8(x[:, :8]))
    sb = mean_agg(pad8(jnp.concatenate(
        [x[:, 8:10], jnp.ones((n, 1), jnp.float32)], axis=1)))
    cnt = jnp.clip(sb[:, 2:3], 1.0, None)
    agg = jnp.concatenate([sa, sb[:, :2]], axis=1) / cnt

    h = x
    for l, p in enumerate(params):
        h = agg @ p["Wl"] + p["bl"] + h @ p["Wr"]
        if l < len(params) - 1:
            h = jax.nn.relu(h)
            agg = mean_agg(pad8(h))[:, :h.shape[1]] / cnt
    return h


# speedup vs baseline: 58.5533x; 3.4244x over previous
import functools

import jax
import jax.numpy as jnp
from jax import lax
from jax.experimental import pallas as pl
from jax.experimental.pallas import tpu as pltpu
from jax.experimental.pallas import tpu_sc as plsc

_NC = 2
_NS = 16
_NW = _NC * _NS
_D = 8
_C = 128
_KB = 8


@functools.cache
def _make_sc_agg(n_nodes: int, n_chunks: int):
    lo = (n_chunks // _NW) & ~7
    extra = (n_chunks - lo * _NW) // 8
    nfull = lo // _KB
    rem = lo - nfull * _KB
    npair = nfull // 2
    odd = nfull - npair * 2
    rps = n_nodes // _NS

    mesh = plsc.VectorSubcoreMesh(core_axis_name="c", subcore_axis_name="s")

    @functools.partial(
        pl.kernel,
        mesh=mesh,
        out_type=jax.ShapeDtypeStruct((_NC, n_nodes, _D), jnp.float32),
        compiler_params=pltpu.CompilerParams(use_tc_tiling_on_sc=False),
        scratch_types=[
            pltpu.VMEM_SHARED((n_nodes, _D), jnp.float32),
            pltpu.VMEM_SHARED((n_nodes, _D), jnp.float32),
            pltpu.VMEM((_KB, _C), jnp.int32),
            pltpu.VMEM((_KB, _C), jnp.int32),
            pltpu.VMEM((_KB, _C), jnp.int32),
            pltpu.VMEM((_KB, _C), jnp.int32),
            pltpu.VMEM((_KB * _C, _D), jnp.float32),
            pltpu.VMEM((_KB * _C, _D), jnp.float32),
            pltpu.SemaphoreType.DMA,
            pltpu.SemaphoreType.DMA,
            pltpu.SemaphoreType.DMA,
            pltpu.SemaphoreType.DMA,
        ],
    )
    def sc_agg(y_hbm, src_hbm, dst_hbm, zeros_hbm, out_hbm,
               ytab, acc, sA, dA, sB, dB, rowsA, rowsB,
               gsA, gsB, ssA, ssB):
        c = lax.axis_index("c")
        s = lax.axis_index("s")
        wid = c * _NS + s

        r0 = s * rps
        pltpu.sync_copy(y_hbm.at[pl.ds(r0, rps), :], ytab.at[pl.ds(r0, rps), :])
        pltpu.sync_copy(zeros_hbm.at[pl.ds(r0, rps), :], acc.at[pl.ds(r0, rps), :])
        plsc.subcore_barrier()

        cb = wid * lo + 8 * jnp.minimum(wid, extra)

        def load_idx(ch0, sref, dref, k):
            pltpu.sync_copy(src_hbm.at[pl.ds(ch0, k), :],
                            sref.at[pl.ds(0, k), :])
            pltpu.sync_copy(dst_hbm.at[pl.ds(ch0, k), :],
                            dref.at[pl.ds(0, k), :])

        def fire_g(sref, rows, sem, k):
            return [pltpu.async_copy(ytab.at[sref.at[j]],
                                     rows.at[pl.ds(j * _C, _C), :], sem)
                    for j in range(k)]

        def fire_s(dref, rows, sem, k):
            return [pltpu.async_copy(rows.at[pl.ds(j * _C, _C), :],
                                     acc.at[dref.at[j]], sem, add=True)
                    for j in range(k)]

        def drain(descs):
            for d in descs:
                d.wait()

        def block(ch0, sref, dref, rows, gsem, ssem, k):
            load_idx(ch0, sref, dref, k)
            drain(fire_g(sref, rows, gsem, k))
            drain(fire_s(dref, rows, ssem, k))

        def pair(i, carry):
            chA = cb + (2 * i) * _KB
            chB = chA + _KB
            load_idx(chA, sA, dA, _KB)
            g_a = fire_g(sA, rowsA, gsA, _KB)
            load_idx(chB, sB, dB, _KB)
            drain(g_a)
            sc_a = fire_s(dA, rowsA, ssA, _KB)
            g_b = fire_g(sB, rowsB, gsB, _KB)
            drain(g_b)
            sc_b = fire_s(dB, rowsB, ssB, _KB)
            drain(sc_a)
            drain(sc_b)
            return carry

        lax.fori_loop(0, npair, pair, 0)
        tail0 = cb + 2 * npair * _KB
        if odd:
            block(tail0, sA, dA, rowsA, gsA, ssA, _KB)
            tail0 = tail0 + _KB
        if rem:
            block(tail0, sB, dB, rowsB, gsB, ssB, rem)
            tail0 = tail0 + rem
        if extra:
            @pl.when(wid < extra)
            def _():
                block(tail0, sA, dA, rowsA, gsA, ssA, 8)

        plsc.subcore_barrier()
        pltpu.sync_copy(acc.at[pl.ds(r0, rps), :],
                        out_hbm.at[c, pl.ds(r0, rps), :])

    return sc_agg


def kernel(x, edge_index, params):
    n = x.shape[0]
    e = edge_index.shape[1]
    npad = -(-n // 128) * 128
    src = edge_index[0].reshape(e // _C, _C)
    dst = edge_index[1].reshape(e // _C, _C)
    sc_agg = _make_sc_agg(npad, e // _C)
    zeros = jnp.zeros((npad, _D), jnp.float32)

    def pad8(m):
        return jnp.pad(m, ((0, npad - n), (0, _D - m.shape[1])))

    def mean_agg(y):
        parts = sc_agg(y, src, dst, zeros)
        return parts[0, :n] + parts[1, :n]

    sa = mean_agg(pad8(x[:, :8]))
    sb = mean_agg(pad8(jnp.concatenate(
        [x[:, 8:10], jnp.ones((n, 1), jnp.float32)], axis=1)))
    cnt = jnp.clip(sb[:, 2:3], 1.0, None)
    agg = jnp.concatenate([sa, sb[:, :2]], axis=1) / cnt

    h = x
    for l, p in enumerate(params):
        h = agg @ p["Wl"] + p["bl"] + h @ p["Wr"]
        if l < len(params) - 1:
            h = jax.nn.relu(h)
            agg = mean_agg(pad8(h))[:, :h.shape[1]] / cnt
    return h


# speedup vs baseline: 58.8142x; 1.0045x over previous
import functools

import jax
import jax.numpy as jnp
from jax import lax
from jax.experimental import pallas as pl
from jax.experimental.pallas import tpu as pltpu
from jax.experimental.pallas import tpu_sc as plsc

_NC = 2
_NS = 16
_NW = _NC * _NS
_D = 8
_C = 128
_KB = 8


@functools.cache
def _make_sc_agg(n_nodes: int, n_chunks: int):
    lo = (n_chunks // _NW) & ~7
    extra = (n_chunks - lo * _NW) // 8
    nfull = lo // _KB
    rem = lo - nfull * _KB
    npair = nfull // 2
    odd = nfull - npair * 2
    rps = n_nodes // _NS

    mesh = plsc.VectorSubcoreMesh(core_axis_name="c", subcore_axis_name="s")

    @functools.partial(
        pl.kernel,
        mesh=mesh,
        out_type=jax.ShapeDtypeStruct((_NC, n_nodes, _D), jnp.float32),
        compiler_params=pltpu.CompilerParams(use_tc_tiling_on_sc=False),
        scratch_types=[
            pltpu.VMEM_SHARED((n_nodes, _D), jnp.float32),
            pltpu.VMEM_SHARED((n_nodes, _D), jnp.float32),
            pltpu.VMEM((_KB * _C,), jnp.int32),
            pltpu.VMEM((_KB * _C,), jnp.int32),
            pltpu.VMEM((_KB * _C,), jnp.int32),
            pltpu.VMEM((_KB * _C,), jnp.int32),
            pltpu.VMEM((_KB * _C, _D), jnp.float32),
            pltpu.VMEM((_KB * _C, _D), jnp.float32),
            pltpu.SemaphoreType.DMA,
            pltpu.SemaphoreType.DMA,
            pltpu.SemaphoreType.DMA,
            pltpu.SemaphoreType.DMA,
        ],
    )
    def sc_agg(y_hbm, src_hbm, dst_hbm, zeros_hbm, out_hbm,
               ytab, acc, sA, dA, sB, dB, rowsA, rowsB,
               gsA, gsB, ssA, ssB):
        c = lax.axis_index("c")
        s = lax.axis_index("s")
        wid = c * _NS + s

        r0 = s * rps
        pltpu.sync_copy(y_hbm.at[pl.ds(r0, rps), :], ytab.at[pl.ds(r0, rps), :])
        pltpu.sync_copy(zeros_hbm.at[pl.ds(r0, rps), :], acc.at[pl.ds(r0, rps), :])
        plsc.subcore_barrier()

        cb = (wid * lo + 8 * jnp.minimum(wid, extra)) * _C

        def load_idx(e0, sref, dref, k):
            pltpu.sync_copy(src_hbm.at[pl.ds(e0, k * _C)],
                            sref.at[pl.ds(0, k * _C)])
            pltpu.sync_copy(dst_hbm.at[pl.ds(e0, k * _C)],
                            dref.at[pl.ds(0, k * _C)])

        def fire_g(sref, rows, sem, k):
            assert k == _KB
            return [pltpu.async_copy(ytab.at[sref], rows, sem)]

        def fire_s(dref, rows, sem, k):
            assert k == _KB
            return [pltpu.async_copy(rows, acc.at[dref], sem, add=True)]

        def drain(descs):
            for d in descs:
                d.wait()

        def block(ch0, sref, dref, rows, gsem, ssem, k):
            load_idx(ch0, sref, dref, k)
            drain(fire_g(sref, rows, gsem, k))
            drain(fire_s(dref, rows, ssem, k))

        assert rem == 0
        eb = _KB * _C

        def pair(i, carry):
            chA = cb + (2 * i) * eb
            chB = chA + eb
            load_idx(chA, sA, dA, _KB)
            g_a = fire_g(sA, rowsA, gsA, _KB)
            load_idx(chB, sB, dB, _KB)
            drain(g_a)
            sc_a = fire_s(dA, rowsA, ssA, _KB)
            g_b = fire_g(sB, rowsB, gsB, _KB)
            drain(g_b)
            sc_b = fire_s(dB, rowsB, ssB, _KB)
            drain(sc_a)
            drain(sc_b)
            return carry

        lax.fori_loop(0, npair, pair, 0)
        tail0 = cb + 2 * npair * eb
        if odd:
            block(tail0, sA, dA, rowsA, gsA, ssA, _KB)
            tail0 = tail0 + eb
        if extra:
            @pl.when(wid < extra)
            def _():
                block(tail0, sB, dB, rowsB, gsB, ssB, _KB)

        plsc.subcore_barrier()
        pltpu.sync_copy(acc.at[pl.ds(r0, rps), :],
                        out_hbm.at[c, pl.ds(r0, rps), :])

    return sc_agg


def kernel(x, edge_index, params):
    n = x.shape[0]
    e = edge_index.shape[1]
    npad = -(-n // 128) * 128
    src = edge_index[0]
    dst = edge_index[1]
    sc_agg = _make_sc_agg(npad, e // _C)
    zeros = jnp.zeros((npad, _D), jnp.float32)

    def pad8(m):
        return jnp.pad(m, ((0, npad - n), (0, _D - m.shape[1])))

    def mean_agg(y):
        parts = sc_agg(y, src, dst, zeros)
        return parts[0, :n] + parts[1, :n]

    sa = mean_agg(pad8(x[:, :8]))
    sb = mean_agg(pad8(jnp.concatenate(
        [x[:, 8:10], jnp.ones((n, 1), jnp.float32)], axis=1)))
    cnt = jnp.clip(sb[:, 2:3], 1.0, None)
    agg = jnp.concatenate([sa, sb[:, :2]], axis=1) / cnt

    h = x
    for l, p in enumerate(params):
        h = agg @ p["Wl"] + p["bl"] + h @ p["Wr"]
        if l < len(params) - 1:
            h = jax.nn.relu(h)
            agg = mean_agg(pad8(h))[:, :h.shape[1]] / cnt
    return h
